# baseline (speedup 1.0000x reference)
import functools

import jax
import jax.numpy as jnp
from jax import lax
from jax.experimental import pallas as pl
from jax.experimental.pallas import tpu as pltpu
from jax.experimental.pallas import tpu_sc as plsc

N_ROWS = 320000
D = 128
N_SEG = 1024
NC = 2
NS = 16
NW = NC * NS
ROWS_PER_W = N_ROWS // NW
CHUNK = 80
NCHUNK = ROWS_PER_W // CHUNK
NBUF = 5
NLANE = 16
NV = D // NLANE
UNROLL = 8


def _sc_partials(X, I32, Z):
    mesh = plsc.VectorSubcoreMesh(core_axis_name="c", subcore_axis_name="s")

    @functools.partial(
        pl.kernel,
        mesh=mesh,
        out_type=jax.ShapeDtypeStruct((NC, N_SEG, D), jnp.float32),
        scratch_types=[
            pltpu.VMEM((NCHUNK, CHUNK), jnp.int32),
            pltpu.VMEM((NBUF, CHUNK, D), jnp.float32),
            pltpu.VMEM((NLANE, D), jnp.float32),
            pltpu.VMEM((NLANE,), jnp.int32),
            pltpu.VMEM_SHARED((N_SEG, D), jnp.float32),
        ] + [pltpu.SemaphoreType.DMA] * NBUF,
    )
    def k(x_hbm, i_hbm, z_hbm, out_hbm, idx_v, data_v, stage_v, sidx_v,
          acc_sh, *sems):
        c = lax.axis_index("c")
        s = lax.axis_index("s")
        wid = c * NS + s
        zf = jnp.zeros((NLANE,), jnp.float32)
        zi = jnp.zeros((NLANE,), jnp.int32)
        lanes = lax.iota(jnp.int32, NLANE)

        def clear_stage():
            for r in range(NLANE):
                for v in range(NV):
                    stage_v[r, pl.ds(NLANE * v, NLANE)] = zf
            sidx_v[...] = zi

        pltpu.sync_copy(z_hbm.at[pl.ds(s * (N_SEG // NS), N_SEG // NS)],
                        acc_sh.at[pl.ds(s * (N_SEG // NS), N_SEG // NS)])
        pltpu.sync_copy(i_hbm.at[wid], idx_v)
        clear_stage()
        plsc.subcore_barrier()

        base = wid * ROWS_PER_W

        def fill(j, b):
            pltpu.async_copy(x_hbm.at[pl.ds(base + j * CHUNK, CHUNK)],
                             data_v.at[b], sems[b])

        def wait_fill(b):
            pltpu.make_async_copy(x_hbm.at[pl.ds(0, CHUNK)], data_v.at[b],
                                  sems[b]).wait()

        def flush():
            pltpu.sync_copy(stage_v, acc_sh.at[sidx_v], add=True)
            clear_stage()

        def append2(lane, rows_a, rows_b, id_a, id_b):
            lane = lax.cond(lane > NLANE - 2,
                            lambda: (flush(), jnp.int32(0))[1],
                            lambda: lane)
            for v in range(NV):
                stage_v[lane, pl.ds(NLANE * v, NLANE)] = rows_a[v]
                stage_v[lane + 1, pl.ds(NLANE * v, NLANE)] = rows_b[v]
            cur = sidx_v[...]
            cur = jnp.where(lanes == lane, id_a, cur)
            cur = jnp.where(lanes == lane + 1, id_b, cur)
            sidx_v[...] = cur
            return lane + 2

        def process(j, b, lane):
            gs = [idx_v[j, pl.ds(NLANE * t, NLANE)] for t in range(CHUNK // NLANE)]
            smin = gs[0][0]
            smax = gs[-1][NLANE - 1]
            one = jnp.ones((NLANE,), jnp.int32)
            zero = jnp.zeros((NLANE,), jnp.int32)
            pc = functools.reduce(
                lambda a, g: a + jnp.where(g == smin, one, zero), gs, zero)
            qc = functools.reduce(
                lambda a, g: a + jnp.where(g == smax, one, zero), gs, zero)
            p = functools.reduce(lambda a, l: a + pc[l], range(NLANE),
                                 jnp.int32(0))
            q = functools.reduce(lambda a, l: a + qc[l], range(NLANE),
                                 jnp.int32(0))

            def rows_at(r):
                return [data_v[b, r, pl.ds(NLANE * v, NLANE)] for v in range(NV)]

            def easy(lane):
                def single(lane):
                    def rbody(r8, acc):
                        for u in range(UNROLL):
                            row = rows_at(r8 * UNROLL + u)
                            acc = tuple(acc[v] + row[v] for v in range(NV))
                        return acc
                    tot = lax.fori_loop(0, CHUNK // UNROLL, rbody, (zf,) * NV)
                    return append2(lane, tot, (zf,) * NV, smin, smin)

                def double(lane):
                    def rbody(r8, carry):
                        pre, tot = carry
                        for u in range(UNROLL):
                            r = r8 * UNROLL + u
                            row = rows_at(r)
                            sel = r < p
                            tot = tuple(tot[v] + row[v] for v in range(NV))
                            pre = tuple(
                                pre[v] + jnp.where(sel, row[v], zf)
                                for v in range(NV))
                        return pre, tot
                    pre, tot = lax.fori_loop(0, CHUNK // UNROLL, rbody,
                                             ((zf,) * NV, (zf,) * NV))
                    suf = tuple(tot[v] - pre[v] for v in range(NV))
                    return append2(lane, pre, suf, smin, smax)

                return lax.cond(p == CHUNK, single, double, lane)

            def hard(lane):
                pltpu.sync_copy(data_v.at[b], acc_sh.at[idx_v.at[j]], add=True)
                return lane

            return lax.cond(p + q >= CHUNK, easy, hard, lane)

        for b in range(NBUF):
            fill(b, b)

        def body(j0, lane):
            for b in range(NBUF):
                j = j0 + b
                wait_fill(b)
                lane = process(j, b, lane)

                @pl.when(j + NBUF < NCHUNK)
                def _():
                    fill(j + NBUF, b)
            return lane

        lax.fori_loop(0, NCHUNK // NBUF,
                      lambda i, ln: body(i * NBUF, ln), jnp.int32(0))
        flush()
        plsc.subcore_barrier()
        pltpu.sync_copy(acc_sh.at[pl.ds(s * (N_SEG // NS), N_SEG // NS)],
                        out_hbm.at[c, pl.ds(s * (N_SEG // NS), N_SEG // NS)])

    return k(X, I32.reshape(NW, NCHUNK, CHUNK), Z)


def _combine(partials):
    def body(p_ref, o_ref):
        o_ref[...] = p_ref[0] + p_ref[1]

    return pl.pallas_call(
        body,
        out_shape=jax.ShapeDtypeStruct((N_SEG, D), jnp.float32),
    )(partials)


def kernel(X, I):
    if I.ndim == 2:
        I = I[:, 0]
    I32 = I.astype(jnp.int32)
    Z = jnp.zeros((N_SEG, D), jnp.float32)
    partials = _sc_partials(X, I32, Z)
    return _combine(partials)


# speedup vs baseline: 1.0208x; 1.0208x over previous
import functools

import jax
import jax.numpy as jnp
from jax import lax
from jax.experimental import pallas as pl
from jax.experimental.pallas import tpu as pltpu
from jax.experimental.pallas import tpu_sc as plsc

N_ROWS = 320000
D = 128
N_SEG = 1024
NC = 2
NS = 16
NW = NC * NS
ROWS_PER_W = N_ROWS // NW
CHUNK = 100
NCHUNK = ROWS_PER_W // CHUNK
SPF = 4
MACRO = CHUNK * SPF
NMACRO = ROWS_PER_W // MACRO
NBUF = 2
SEG_PER_TILE = N_SEG // NS


def _sc_partials(X, I32, Z):
    mesh = plsc.VectorSubcoreMesh(core_axis_name="c", subcore_axis_name="s")

    @functools.partial(
        pl.kernel,
        mesh=mesh,
        out_type=jax.ShapeDtypeStruct((NC, N_SEG, D), jnp.float32),
        scratch_types=[
            pltpu.VMEM((NCHUNK, CHUNK), jnp.int32),
            pltpu.VMEM((NBUF, MACRO, D), jnp.float32),
            pltpu.VMEM_SHARED((N_SEG, D), jnp.float32),
        ] + [pltpu.SemaphoreType.DMA] * NBUF,
    )
    def k(x_hbm, i_hbm, z_hbm, out_hbm, idx_v, data_v, acc_sh, *sems):
        c = lax.axis_index("c")
        s = lax.axis_index("s")
        wid = c * NS + s
        base = wid * ROWS_PER_W

        def fill(m, b):
            pltpu.async_copy(x_hbm.at[pl.ds(base + m * MACRO, MACRO)],
                             data_v.at[b], sems[b])

        def wait_fill(b):
            pltpu.make_async_copy(x_hbm.at[pl.ds(0, MACRO)], data_v.at[b],
                                  sems[b]).wait()

        for b in range(NBUF):
            fill(b, b)
        pltpu.sync_copy(z_hbm.at[pl.ds(s * SEG_PER_TILE, SEG_PER_TILE)],
                        acc_sh.at[pl.ds(s * SEG_PER_TILE, SEG_PER_TILE)])
        pltpu.sync_copy(i_hbm.at[wid], idx_v)
        plsc.subcore_barrier()

        def drain(m, b):
            for t in range(SPF):
                pltpu.sync_copy(data_v.at[b, pl.ds(t * CHUNK, CHUNK)],
                                acc_sh.at[idx_v.at[m * SPF + t]], add=True)

        def body(m2, carry):
            for b in range(NBUF):
                m = m2 * NBUF + b
                wait_fill(b)
                drain(m, b)

                @pl.when(m + NBUF < NMACRO)
                def _():
                    fill(m + NBUF, b)
            return carry

        lax.fori_loop(0, NMACRO // NBUF, body, 0)
        wait_fill(0)
        drain(NMACRO - 1, 0)
        plsc.subcore_barrier()
        pltpu.sync_copy(acc_sh.at[pl.ds(s * SEG_PER_TILE, SEG_PER_TILE)],
                        out_hbm.at[c, pl.ds(s * SEG_PER_TILE, SEG_PER_TILE)])

    return k(X, I32.reshape(NW, NCHUNK, CHUNK), Z)


def _combine(partials):
    def body(p_ref, o_ref):
        o_ref[...] = p_ref[0] + p_ref[1]

    return pl.pallas_call(
        body,
        out_shape=jax.ShapeDtypeStruct((N_SEG, D), jnp.float32),
    )(partials)


def kernel(X, I):
    if I.ndim == 2:
        I = I[:, 0]
    I32 = I.astype(jnp.int32)
    Z = jnp.zeros((N_SEG, D), jnp.float32)
    partials = _sc_partials(X, I32, Z)
    return _combine(partials)


# speedup vs baseline: 1.1119x; 1.0892x over previous
import functools

import jax
import jax.numpy as jnp
from jax import lax
from jax.experimental import pallas as pl
from jax.experimental.pallas import tpu as pltpu
from jax.experimental.pallas import tpu_sc as plsc

N_ROWS = 320000
D = 128
N_SEG = 1024
NC = 2
NS = 16
NW = NC * NS
ROWS_PER_W = N_ROWS // NW
CHUNK = 80
NCHUNK = ROWS_PER_W // CHUNK
SPF = 1
MACRO = CHUNK * SPF
NMACRO = ROWS_PER_W // MACRO
NBUF = 5
SEG_PER_TILE = N_SEG // NS


def _sc_partials(X, I32, Z):
    mesh = plsc.VectorSubcoreMesh(core_axis_name="c", subcore_axis_name="s")

    @functools.partial(
        pl.kernel,
        mesh=mesh,
        out_type=jax.ShapeDtypeStruct((NC, N_SEG, D), jnp.float32),
        scratch_types=[
            pltpu.VMEM((NCHUNK, CHUNK), jnp.int32),
            pltpu.VMEM((NBUF, MACRO, D), jnp.float32),
            pltpu.VMEM_SHARED((N_SEG, D), jnp.float32),
        ] + [pltpu.SemaphoreType.DMA] * NBUF,
    )
    def k(x_hbm, i_hbm, z_hbm, out_hbm, idx_v, data_v, acc_sh, *sems):
        c = lax.axis_index("c")
        s = lax.axis_index("s")
        wid = c * NS + s
        base = wid * ROWS_PER_W

        def fill(m, b):
            pltpu.async_copy(x_hbm.at[pl.ds(base + m * MACRO, MACRO)],
                             data_v.at[b], sems[b])

        def wait_fill(b):
            pltpu.make_async_copy(x_hbm.at[pl.ds(0, MACRO)], data_v.at[b],
                                  sems[b]).wait()

        for b in range(NBUF):
            fill(b, b)
        pltpu.sync_copy(z_hbm.at[pl.ds(s * SEG_PER_TILE, SEG_PER_TILE)],
                        acc_sh.at[pl.ds(s * SEG_PER_TILE, SEG_PER_TILE)])
        pltpu.sync_copy(i_hbm.at[wid], idx_v)
        plsc.subcore_barrier()

        def drain(m, b):
            for t in range(SPF):
                pltpu.sync_copy(data_v.at[b, pl.ds(t * CHUNK, CHUNK)],
                                acc_sh.at[idx_v.at[m * SPF + t]], add=True)

        def body(m2, carry):
            for b in range(NBUF):
                m = m2 * NBUF + b
                wait_fill(b)
                drain(m, b)

                @pl.when(m + NBUF < NMACRO)
                def _():
                    fill(m + NBUF, b)
            return carry

        lax.fori_loop(0, NMACRO // NBUF, body, 0)
        plsc.subcore_barrier()
        pltpu.sync_copy(acc_sh.at[pl.ds(s * SEG_PER_TILE, SEG_PER_TILE)],
                        out_hbm.at[c, pl.ds(s * SEG_PER_TILE, SEG_PER_TILE)])

    return k(X, I32.reshape(NW, NCHUNK, CHUNK), Z)


def _combine(partials):
    def body(p_ref, o_ref):
        o_ref[...] = p_ref[0] + p_ref[1]

    return pl.pallas_call(
        body,
        out_shape=jax.ShapeDtypeStruct((N_SEG, D), jnp.float32),
    )(partials)


def kernel(X, I):
    if I.ndim == 2:
        I = I[:, 0]
    I32 = I.astype(jnp.int32)
    Z = jnp.zeros((N_SEG, D), jnp.float32)
    partials = _sc_partials(X, I32, Z)
    return _combine(partials)


# speedup vs baseline: 1.1736x; 1.0555x over previous
import functools

import jax
import jax.numpy as jnp
from jax import lax
from jax.experimental import pallas as pl
from jax.experimental.pallas import tpu as pltpu
from jax.experimental.pallas import tpu_sc as plsc

N_ROWS = 320000
D = 128
N_SEG = 1024
NC = 2
NS = 16
NW = NC * NS
ROWS_PER_W = N_ROWS // NW
CHUNK = 80
NCHUNK = ROWS_PER_W // CHUNK
SPF = 1
MACRO = CHUNK * SPF
NMACRO = ROWS_PER_W // MACRO
NBUF = 5
SEG_PER_TILE = N_SEG // NS


def _sc_partials(X, I32, Z):
    mesh = plsc.VectorSubcoreMesh(core_axis_name="c", subcore_axis_name="s")

    @functools.partial(
        pl.kernel,
        mesh=mesh,
        out_type=jax.ShapeDtypeStruct((NC, N_SEG, D), jnp.float32),
        scratch_types=[
            pltpu.VMEM((NCHUNK, CHUNK), jnp.int32),
            pltpu.VMEM((NBUF, MACRO, D), jnp.float32),
            pltpu.VMEM_SHARED((N_SEG, D), jnp.float32),
        ] + [pltpu.SemaphoreType.DMA] * NBUF,
    )
    def k(x_hbm, i_hbm, z_hbm, out_hbm, idx_v, data_v, acc_sh, *sems):
        c = lax.axis_index("c")
        s = lax.axis_index("s")
        wid = c * NS + s
        base = wid * ROWS_PER_W

        def fill(m, b):
            pltpu.async_copy(x_hbm.at[pl.ds(base + m * MACRO, MACRO)],
                             data_v.at[b], sems[b])

        def wait_fill(b):
            pltpu.make_async_copy(x_hbm.at[pl.ds(0, MACRO)], data_v.at[b],
                                  sems[b]).wait()

        pltpu.sync_copy(z_hbm.at[pl.ds(s * SEG_PER_TILE, SEG_PER_TILE)],
                        acc_sh.at[pl.ds(s * SEG_PER_TILE, SEG_PER_TILE)])
        pltpu.sync_copy(i_hbm.at[wid], idx_v)
        plsc.subcore_barrier()
        for b in range(NBUF):
            fill(b, b)

        def drain(m, b):
            for t in range(SPF):
                pltpu.sync_copy(data_v.at[b, pl.ds(t * CHUNK, CHUNK)],
                                acc_sh.at[idx_v.at[m * SPF + t]], add=True)

        def body(m2, carry):
            for b in range(NBUF):
                m = m2 * NBUF + b
                wait_fill(b)
                drain(m, b)

                @pl.when(m + NBUF < NMACRO)
                def _():
                    fill(m + NBUF, b)
            return carry

        lax.fori_loop(0, NMACRO // NBUF, body, 0)
        plsc.subcore_barrier()
        pltpu.sync_copy(acc_sh.at[pl.ds(s * SEG_PER_TILE, SEG_PER_TILE)],
                        out_hbm.at[c, pl.ds(s * SEG_PER_TILE, SEG_PER_TILE)])

    return k(X, I32.reshape(NW, NCHUNK, CHUNK), Z)


def _combine(partials):
    def body(p_ref, o_ref):
        o_ref[...] = p_ref[0] + p_ref[1]

    return pl.pallas_call(
        body,
        out_shape=jax.ShapeDtypeStruct((N_SEG, D), jnp.float32),
    )(partials)


def kernel(X, I):
    if I.ndim == 2:
        I = I[:, 0]
    I32 = I.astype(jnp.int32)
    Z = jnp.zeros((N_SEG, D), jnp.float32)
    partials = _sc_partials(X, I32, Z)
    return _combine(partials)


# speedup vs baseline: 1.2478x; 1.0633x over previous
import functools

import jax
import jax.numpy as jnp
from jax import lax
from jax.experimental import pallas as pl
from jax.experimental.pallas import tpu as pltpu
from jax.experimental.pallas import tpu_sc as plsc

N_ROWS = 320000
D = 128
N_SEG = 1024
NC = 2
NS = 16
NW = NC * NS
ROWS_PER_W = N_ROWS // NW
CHUNK = 80
NCHUNK = ROWS_PER_W // CHUNK
NBUF = 5
SEG_PER_TILE = N_SEG // NS
NLANE = 16
NV = D // NLANE
UNROLL = 8
TEC_B = (1, 3)


def _sc_partials(X, I32, Z):
    mesh = plsc.VectorSubcoreMesh(core_axis_name="c", subcore_axis_name="s")

    @functools.partial(
        pl.kernel,
        mesh=mesh,
        out_type=jax.ShapeDtypeStruct((NC, N_SEG, D), jnp.float32),
        scratch_types=[
            pltpu.VMEM((NCHUNK, CHUNK), jnp.int32),
            pltpu.VMEM((NBUF, CHUNK, D), jnp.float32),
            pltpu.VMEM((NLANE, D), jnp.float32),
            pltpu.VMEM((NLANE,), jnp.int32),
            pltpu.VMEM_SHARED((N_SEG, D), jnp.float32),
        ] + [pltpu.SemaphoreType.DMA] * (2 * NBUF),
    )
    def k(x_hbm, i_hbm, z_hbm, out_hbm, idx_v, data_v, stage_v, sidx_v,
          acc_sh, *sems):
        c = lax.axis_index("c")
        s = lax.axis_index("s")
        wid = c * NS + s
        base = wid * ROWS_PER_W
        zf = jnp.zeros((NLANE,), jnp.float32)
        zi = jnp.zeros((NLANE,), jnp.int32)
        lanes = lax.iota(jnp.int32, NLANE)

        def clear_stage():
            for r in range(NLANE):
                for v in range(NV):
                    stage_v[r, pl.ds(NLANE * v, NLANE)] = zf
            sidx_v[...] = zi

        pltpu.sync_copy(z_hbm.at[pl.ds(s * SEG_PER_TILE, SEG_PER_TILE)],
                        acc_sh.at[pl.ds(s * SEG_PER_TILE, SEG_PER_TILE)])
        pltpu.sync_copy(i_hbm.at[wid], idx_v)
        clear_stage()
        plsc.subcore_barrier()

        def fill(j, b):
            pltpu.async_copy(x_hbm.at[pl.ds(base + j * CHUNK, CHUNK)],
                             data_v.at[b], sems[b])

        def wait_fill(b):
            pltpu.make_async_copy(x_hbm.at[pl.ds(0, CHUNK)], data_v.at[b],
                                  sems[b]).wait()

        def scatter(j, b):
            pltpu.async_copy(data_v.at[b], acc_sh.at[idx_v.at[j]],
                             sems[NBUF + b], add=True)

        def wait_scatter(b):
            pltpu.make_async_copy(x_hbm.at[pl.ds(0, CHUNK)], data_v.at[b],
                                  sems[NBUF + b]).wait()

        def first_last(j):
            return (idx_v[j, pl.ds(0, NLANE)][0],
                    idx_v[j, pl.ds(CHUNK - NLANE, NLANE)][NLANE - 1])

        def flush():
            pltpu.sync_copy(stage_v, acc_sh.at[sidx_v], add=True)
            clear_stage()

        def append1(lane, rows, seg):
            lane = lax.cond(lane > NLANE - 1,
                            lambda: (flush(), jnp.int32(0))[1],
                            lambda: lane)
            for v in range(NV):
                stage_v[lane, pl.ds(NLANE * v, NLANE)] = rows[v]
            sidx_v[...] = jnp.where(lanes == lane, seg, sidx_v[...])
            return lane + 1

        def reduce_chunk(j, b, lane, seg):
            def rbody(r8, acc):
                for u in range(UNROLL):
                    r = r8 * UNROLL + u
                    acc = tuple(
                        acc[v] + data_v[b, r, pl.ds(NLANE * v, NLANE)]
                        for v in range(NV))
                return acc
            tot = lax.fori_loop(0, CHUNK // UNROLL, rbody, (zf,) * NV)
            return append1(lane, tot, seg)

        def retire(j, b):
            if b in TEC_B:
                sm, sx = first_last(j)

                @pl.when(sm != sx)
                def _():
                    wait_scatter(b)
            else:
                wait_scatter(b)

        def step(j, b, bp):
            @pl.when(j > 0)
            def _():
                retire(j - 1, bp)

                @pl.when(j - 1 + NBUF < NCHUNK)
                def _():
                    fill(j - 1 + NBUF, bp)

            wait_fill(b)
            return j, b

        def body(j0, lane):
            for b in range(NBUF):
                j = j0 + b
                bp = (b - 1) % NBUF
                step(j, b, bp)
                if b in TEC_B:
                    sm, sx = first_last(j)

                    def tec_path(lane, j=j, b=b, sm=sm):
                        return reduce_chunk(j, b, lane, sm)

                    def dma_path(lane, j=j, b=b):
                        scatter(j, b)
                        return lane

                    lane = lax.cond(sm == sx, tec_path, dma_path, lane)
                else:
                    scatter(j, b)
            return lane

        for b in range(NBUF):
            fill(b, b)
        lax.fori_loop(0, NCHUNK // NBUF,
                      lambda i, ln: body(i * NBUF, ln), jnp.int32(0))
        retire(NCHUNK - 1, (NCHUNK - 1) % NBUF)
        flush()
        plsc.subcore_barrier()
        pltpu.sync_copy(acc_sh.at[pl.ds(s * SEG_PER_TILE, SEG_PER_TILE)],
                        out_hbm.at[c, pl.ds(s * SEG_PER_TILE, SEG_PER_TILE)])

    return k(X, I32.reshape(NW, NCHUNK, CHUNK), Z)


def _combine(partials):
    def body(p_ref, o_ref):
        o_ref[...] = p_ref[0] + p_ref[1]

    return pl.pallas_call(
        body,
        out_shape=jax.ShapeDtypeStruct((N_SEG, D), jnp.float32),
    )(partials)


def kernel(X, I):
    if I.ndim == 2:
        I = I[:, 0]
    I32 = I.astype(jnp.int32)
    Z = jnp.zeros((N_SEG, D), jnp.float32)
    partials = _sc_partials(X, I32, Z)
    return _combine(partials)


# speedup vs baseline: 1.4383x; 1.1526x over previous
import functools

import jax
import jax.numpy as jnp
from jax import lax
from jax.experimental import pallas as pl
from jax.experimental.pallas import tpu as pltpu
from jax.experimental.pallas import tpu_sc as plsc

N_ROWS = 320000
D = 128
N_SEG = 1024
NC = 2
NS = 16
NW = NC * NS
ROWS_PER_W = N_ROWS // NW
CHUNK = 80
NCHUNK = ROWS_PER_W // CHUNK
NBUF = 5
SEG_PER_TILE = N_SEG // NS
NLANE = 16
NV = D // NLANE
UNROLL = 8
TEC_B = (1, 2, 3)


def _sc_partials(X, I32, Z):
    mesh = plsc.VectorSubcoreMesh(core_axis_name="c", subcore_axis_name="s")

    @functools.partial(
        pl.kernel,
        mesh=mesh,
        out_type=jax.ShapeDtypeStruct((NC, N_SEG, D), jnp.float32),
        scratch_types=[
            pltpu.VMEM((NCHUNK, CHUNK), jnp.int32),
            pltpu.VMEM((NBUF, CHUNK, D), jnp.float32),
            pltpu.VMEM((NLANE, D), jnp.float32),
            pltpu.VMEM((NLANE,), jnp.int32),
            pltpu.VMEM_SHARED((N_SEG, D), jnp.float32),
        ] + [pltpu.SemaphoreType.DMA] * (2 * NBUF),
    )
    def k(x_hbm, i_hbm, z_hbm, out_hbm, idx_v, data_v, stage_v, sidx_v,
          acc_sh, *sems):
        c = lax.axis_index("c")
        s = lax.axis_index("s")
        wid = c * NS + s
        base = wid * ROWS_PER_W
        zf = jnp.zeros((NLANE,), jnp.float32)
        zi = jnp.zeros((NLANE,), jnp.int32)
        lanes = lax.iota(jnp.int32, NLANE)

        def clear_stage():
            for r in range(NLANE):
                for v in range(NV):
                    stage_v[r, pl.ds(NLANE * v, NLANE)] = zf
            sidx_v[...] = zi

        pltpu.sync_copy(z_hbm.at[pl.ds(s * SEG_PER_TILE, SEG_PER_TILE)],
                        acc_sh.at[pl.ds(s * SEG_PER_TILE, SEG_PER_TILE)])
        pltpu.sync_copy(i_hbm.at[wid], idx_v)
        clear_stage()
        plsc.subcore_barrier()

        def fill(j, b):
            pltpu.async_copy(x_hbm.at[pl.ds(base + j * CHUNK, CHUNK)],
                             data_v.at[b], sems[b])

        def wait_fill(b):
            pltpu.make_async_copy(x_hbm.at[pl.ds(0, CHUNK)], data_v.at[b],
                                  sems[b]).wait()

        def scatter(j, b):
            pltpu.async_copy(data_v.at[b], acc_sh.at[idx_v.at[j]],
                             sems[NBUF + b], add=True)

        def wait_scatter(b):
            pltpu.make_async_copy(x_hbm.at[pl.ds(0, CHUNK)], data_v.at[b],
                                  sems[NBUF + b]).wait()

        def first_last(j):
            return (idx_v[j, pl.ds(0, NLANE)][0],
                    idx_v[j, pl.ds(CHUNK - NLANE, NLANE)][NLANE - 1])

        def flush():
            pltpu.sync_copy(stage_v, acc_sh.at[sidx_v], add=True)
            clear_stage()

        def append1(lane, rows, seg):
            lane = lax.cond(lane > NLANE - 1,
                            lambda: (flush(), jnp.int32(0))[1],
                            lambda: lane)
            for v in range(NV):
                stage_v[lane, pl.ds(NLANE * v, NLANE)] = rows[v]
            sidx_v[...] = jnp.where(lanes == lane, seg, sidx_v[...])
            return lane + 1

        def reduce_chunk(j, b, lane, seg):
            def rbody(r8, acc):
                for u in range(UNROLL):
                    r = r8 * UNROLL + u
                    acc = tuple(
                        acc[v] + data_v[b, r, pl.ds(NLANE * v, NLANE)]
                        for v in range(NV))
                return acc
            tot = lax.fori_loop(0, CHUNK // UNROLL, rbody, (zf,) * NV)
            return append1(lane, tot, seg)

        def retire(j, b):
            if b in TEC_B:
                sm, sx = first_last(j)

                @pl.when(sm != sx)
                def _():
                    wait_scatter(b)
            else:
                wait_scatter(b)

        def step(j, b, bp):
            @pl.when(j > 0)
            def _():
                retire(j - 1, bp)

                @pl.when(j - 1 + NBUF < NCHUNK)
                def _():
                    fill(j - 1 + NBUF, bp)

            wait_fill(b)
            return j, b

        def body(j0, lane):
            for b in range(NBUF):
                j = j0 + b
                bp = (b - 1) % NBUF
                step(j, b, bp)
                if b in TEC_B:
                    sm, sx = first_last(j)

                    def tec_path(lane, j=j, b=b, sm=sm):
                        return reduce_chunk(j, b, lane, sm)

                    def dma_path(lane, j=j, b=b):
                        scatter(j, b)
                        return lane

                    lane = lax.cond(sm == sx, tec_path, dma_path, lane)
                else:
                    scatter(j, b)
            return lane

        for b in range(NBUF):
            fill(b, b)
        lax.fori_loop(0, NCHUNK // NBUF,
                      lambda i, ln: body(i * NBUF, ln), jnp.int32(0))
        retire(NCHUNK - 1, (NCHUNK - 1) % NBUF)
        flush()
        plsc.subcore_barrier()
        pltpu.sync_copy(acc_sh.at[pl.ds(s * SEG_PER_TILE, SEG_PER_TILE)],
                        out_hbm.at[c, pl.ds(s * SEG_PER_TILE, SEG_PER_TILE)])

    return k(X, I32.reshape(NW, NCHUNK, CHUNK), Z)


def _combine(partials):
    def body(p_ref, o_ref):
        o_ref[...] = p_ref[0] + p_ref[1]

    return pl.pallas_call(
        body,
        out_shape=jax.ShapeDtypeStruct((N_SEG, D), jnp.float32),
    )(partials)


def kernel(X, I):
    if I.ndim == 2:
        I = I[:, 0]
    I32 = I.astype(jnp.int32)
    Z = jnp.zeros((N_SEG, D), jnp.float32)
    partials = _sc_partials(X, I32, Z)
    return _combine(partials)


# speedup vs baseline: 1.5735x; 1.0940x over previous
import functools

import jax
import jax.numpy as jnp
from jax import lax
from jax.experimental import pallas as pl
from jax.experimental.pallas import tpu as pltpu
from jax.experimental.pallas import tpu_sc as plsc

N_ROWS = 320000
D = 128
N_SEG = 1024
NC = 2
NS = 16
NW = NC * NS
ROWS_PER_W = N_ROWS // NW
CHUNK = 80
NCHUNK = ROWS_PER_W // CHUNK
NBUF = 5
SEG_PER_TILE = N_SEG // NS
NLANE = 16
NV = D // NLANE
UNROLL = 8
TEC_B = (1, 2, 3, 4)


def _sc_partials(X, I32, Z):
    mesh = plsc.VectorSubcoreMesh(core_axis_name="c", subcore_axis_name="s")

    @functools.partial(
        pl.kernel,
        mesh=mesh,
        out_type=jax.ShapeDtypeStruct((NC, N_SEG, D), jnp.float32),
        scratch_types=[
            pltpu.VMEM((NCHUNK, CHUNK), jnp.int32),
            pltpu.VMEM((NBUF, CHUNK, D), jnp.float32),
            pltpu.VMEM((NLANE, D), jnp.float32),
            pltpu.VMEM((NLANE,), jnp.int32),
            pltpu.VMEM_SHARED((N_SEG, D), jnp.float32),
        ] + [pltpu.SemaphoreType.DMA] * (2 * NBUF),
    )
    def k(x_hbm, i_hbm, z_hbm, out_hbm, idx_v, data_v, stage_v, sidx_v,
          acc_sh, *sems):
        c = lax.axis_index("c")
        s = lax.axis_index("s")
        wid = c * NS + s
        base = wid * ROWS_PER_W
        zf = jnp.zeros((NLANE,), jnp.float32)
        zi = jnp.zeros((NLANE,), jnp.int32)
        lanes = lax.iota(jnp.int32, NLANE)

        def clear_stage():
            for r in range(NLANE):
                for v in range(NV):
                    stage_v[r, pl.ds(NLANE * v, NLANE)] = zf
            sidx_v[...] = zi

        pltpu.sync_copy(z_hbm.at[pl.ds(s * SEG_PER_TILE, SEG_PER_TILE)],
                        acc_sh.at[pl.ds(s * SEG_PER_TILE, SEG_PER_TILE)])
        pltpu.sync_copy(i_hbm.at[wid], idx_v)
        clear_stage()
        plsc.subcore_barrier()

        def fill(j, b):
            pltpu.async_copy(x_hbm.at[pl.ds(base + j * CHUNK, CHUNK)],
                             data_v.at[b], sems[b])

        def wait_fill(b):
            pltpu.make_async_copy(x_hbm.at[pl.ds(0, CHUNK)], data_v.at[b],
                                  sems[b]).wait()

        def scatter(j, b):
            pltpu.async_copy(data_v.at[b], acc_sh.at[idx_v.at[j]],
                             sems[NBUF + b], add=True)

        def wait_scatter(b):
            pltpu.make_async_copy(x_hbm.at[pl.ds(0, CHUNK)], data_v.at[b],
                                  sems[NBUF + b]).wait()

        def first_last(j):
            return (idx_v[j, pl.ds(0, NLANE)][0],
                    idx_v[j, pl.ds(CHUNK - NLANE, NLANE)][NLANE - 1])

        def flush():
            pltpu.sync_copy(stage_v, acc_sh.at[sidx_v], add=True)
            clear_stage()

        def append1(lane, rows, seg):
            lane = lax.cond(lane > NLANE - 1,
                            lambda: (flush(), jnp.int32(0))[1],
                            lambda: lane)
            for v in range(NV):
                stage_v[lane, pl.ds(NLANE * v, NLANE)] = rows[v]
            sidx_v[...] = jnp.where(lanes == lane, seg, sidx_v[...])
            return lane + 1

        def reduce_chunk(j, b, lane, seg):
            def rbody(r8, acc):
                for u in range(UNROLL):
                    r = r8 * UNROLL + u
                    acc = tuple(
                        acc[v] + data_v[b, r, pl.ds(NLANE * v, NLANE)]
                        for v in range(NV))
                return acc
            tot = lax.fori_loop(0, CHUNK // UNROLL, rbody, (zf,) * NV)
            return append1(lane, tot, seg)

        def retire(j, b):
            if b in TEC_B:
                sm, sx = first_last(j)

                @pl.when(sm != sx)
                def _():
                    wait_scatter(b)
            else:
                wait_scatter(b)

        def step(j, b, bp):
            @pl.when(j > 0)
            def _():
                retire(j - 1, bp)

                @pl.when(j - 1 + NBUF < NCHUNK)
                def _():
                    fill(j - 1 + NBUF, bp)

            wait_fill(b)
            return j, b

        def body(j0, lane):
            for b in range(NBUF):
                j = j0 + b
                bp = (b - 1) % NBUF
                step(j, b, bp)
                if b in TEC_B:
                    sm, sx = first_last(j)

                    def tec_path(lane, j=j, b=b, sm=sm):
                        return reduce_chunk(j, b, lane, sm)

                    def dma_path(lane, j=j, b=b):
                        scatter(j, b)
                        return lane

                    lane = lax.cond(sm == sx, tec_path, dma_path, lane)
                else:
                    scatter(j, b)
            return lane

        for b in range(NBUF):
            fill(b, b)
        lax.fori_loop(0, NCHUNK // NBUF,
                      lambda i, ln: body(i * NBUF, ln), jnp.int32(0))
        retire(NCHUNK - 1, (NCHUNK - 1) % NBUF)
        flush()
        plsc.subcore_barrier()
        pltpu.sync_copy(acc_sh.at[pl.ds(s * SEG_PER_TILE, SEG_PER_TILE)],
                        out_hbm.at[c, pl.ds(s * SEG_PER_TILE, SEG_PER_TILE)])

    return k(X, I32.reshape(NW, NCHUNK, CHUNK), Z)


def _combine(partials):
    def body(p_ref, o_ref):
        o_ref[...] = p_ref[0] + p_ref[1]

    return pl.pallas_call(
        body,
        out_shape=jax.ShapeDtypeStruct((N_SEG, D), jnp.float32),
    )(partials)


def kernel(X, I):
    if I.ndim == 2:
        I = I[:, 0]
    I32 = I.astype(jnp.int32)
    Z = jnp.zeros((N_SEG, D), jnp.float32)
    partials = _sc_partials(X, I32, Z)
    return _combine(partials)


# speedup vs baseline: 1.8046x; 1.1469x over previous
import functools

import jax
import jax.numpy as jnp
from jax import lax
from jax.experimental import pallas as pl
from jax.experimental.pallas import tpu as pltpu
from jax.experimental.pallas import tpu_sc as plsc

N_ROWS = 320000
D = 128
N_SEG = 1024
NC = 2
NS = 16
NW = NC * NS
ROWS_PER_W = N_ROWS // NW
CHUNK = 80
NCHUNK = ROWS_PER_W // CHUNK
NBUF = 5
SEG_PER_TILE = N_SEG // NS
NLANE = 16
NV = D // NLANE
UNROLL = 8
TEC_B = (0, 1, 2, 3, 4)


def _sc_partials(X, I32, Z):
    mesh = plsc.VectorSubcoreMesh(core_axis_name="c", subcore_axis_name="s")

    @functools.partial(
        pl.kernel,
        mesh=mesh,
        out_type=jax.ShapeDtypeStruct((NC, N_SEG, D), jnp.float32),
        scratch_types=[
            pltpu.VMEM((NCHUNK, CHUNK), jnp.int32),
            pltpu.VMEM((NBUF, CHUNK, D), jnp.float32),
            pltpu.VMEM((NLANE, D), jnp.float32),
            pltpu.VMEM((NLANE,), jnp.int32),
            pltpu.VMEM_SHARED((N_SEG, D), jnp.float32),
        ] + [pltpu.SemaphoreType.DMA] * (2 * NBUF),
    )
    def k(x_hbm, i_hbm, z_hbm, out_hbm, idx_v, data_v, stage_v, sidx_v,
          acc_sh, *sems):
        c = lax.axis_index("c")
        s = lax.axis_index("s")
        wid = c * NS + s
        base = wid * ROWS_PER_W
        zf = jnp.zeros((NLANE,), jnp.float32)
        zi = jnp.zeros((NLANE,), jnp.int32)
        lanes = lax.iota(jnp.int32, NLANE)

        def clear_stage():
            for r in range(NLANE):
                for v in range(NV):
                    stage_v[r, pl.ds(NLANE * v, NLANE)] = zf
            sidx_v[...] = zi

        pltpu.sync_copy(z_hbm.at[pl.ds(s * SEG_PER_TILE, SEG_PER_TILE)],
                        acc_sh.at[pl.ds(s * SEG_PER_TILE, SEG_PER_TILE)])
        pltpu.sync_copy(i_hbm.at[wid], idx_v)
        clear_stage()
        plsc.subcore_barrier()

        def fill(j, b):
            pltpu.async_copy(x_hbm.at[pl.ds(base + j * CHUNK, CHUNK)],
                             data_v.at[b], sems[b])

        def wait_fill(b):
            pltpu.make_async_copy(x_hbm.at[pl.ds(0, CHUNK)], data_v.at[b],
                                  sems[b]).wait()

        def scatter(j, b):
            pltpu.async_copy(data_v.at[b], acc_sh.at[idx_v.at[j]],
                             sems[NBUF + b], add=True)

        def wait_scatter(b):
            pltpu.make_async_copy(x_hbm.at[pl.ds(0, CHUNK)], data_v.at[b],
                                  sems[NBUF + b]).wait()

        def first_last(j):
            return (idx_v[j, pl.ds(0, NLANE)][0],
                    idx_v[j, pl.ds(CHUNK - NLANE, NLANE)][NLANE - 1])

        def flush():
            pltpu.sync_copy(stage_v, acc_sh.at[sidx_v], add=True)
            clear_stage()

        def append1(lane, rows, seg):
            lane = lax.cond(lane > NLANE - 1,
                            lambda: (flush(), jnp.int32(0))[1],
                            lambda: lane)
            for v in range(NV):
                stage_v[lane, pl.ds(NLANE * v, NLANE)] = rows[v]
            sidx_v[...] = jnp.where(lanes == lane, seg, sidx_v[...])
            return lane + 1

        def reduce_chunk(j, b, lane, seg):
            def rbody(r8, acc):
                for u in range(UNROLL):
                    r = r8 * UNROLL + u
                    acc = tuple(
                        acc[v] + data_v[b, r, pl.ds(NLANE * v, NLANE)]
                        for v in range(NV))
                return acc
            tot = lax.fori_loop(0, CHUNK // UNROLL, rbody, (zf,) * NV)
            return append1(lane, tot, seg)

        def retire(j, b):
            if b in TEC_B:
                sm, sx = first_last(j)

                @pl.when(sm != sx)
                def _():
                    wait_scatter(b)
            else:
                wait_scatter(b)

        def step(j, b, bp):
            @pl.when(j > 0)
            def _():
                retire(j - 1, bp)

                @pl.when(j - 1 + NBUF < NCHUNK)
                def _():
                    fill(j - 1 + NBUF, bp)

            wait_fill(b)
            return j, b

        def body(j0, lane):
            for b in range(NBUF):
                j = j0 + b
                bp = (b - 1) % NBUF
                step(j, b, bp)
                if b in TEC_B:
                    sm, sx = first_last(j)

                    def tec_path(lane, j=j, b=b, sm=sm):
                        return reduce_chunk(j, b, lane, sm)

                    def dma_path(lane, j=j, b=b):
                        scatter(j, b)
                        return lane

                    lane = lax.cond(sm == sx, tec_path, dma_path, lane)
                else:
                    scatter(j, b)
            return lane

        for b in range(NBUF):
            fill(b, b)
        lax.fori_loop(0, NCHUNK // NBUF,
                      lambda i, ln: body(i * NBUF, ln), jnp.int32(0))
        retire(NCHUNK - 1, (NCHUNK - 1) % NBUF)
        flush()
        plsc.subcore_barrier()
        pltpu.sync_copy(acc_sh.at[pl.ds(s * SEG_PER_TILE, SEG_PER_TILE)],
                        out_hbm.at[c, pl.ds(s * SEG_PER_TILE, SEG_PER_TILE)])

    return k(X, I32.reshape(NW, NCHUNK, CHUNK), Z)


def _combine(partials):
    def body(p_ref, o_ref):
        o_ref[...] = p_ref[0] + p_ref[1]

    return pl.pallas_call(
        body,
        out_shape=jax.ShapeDtypeStruct((N_SEG, D), jnp.float32),
    )(partials)


def kernel(X, I):
    if I.ndim == 2:
        I = I[:, 0]
    I32 = I.astype(jnp.int32)
    Z = jnp.zeros((N_SEG, D), jnp.float32)
    partials = _sc_partials(X, I32, Z)
    return _combine(partials)
